# baseline (speedup 1.0000x reference)
import functools

import jax
import jax.numpy as jnp
from jax import lax
from jax.experimental import pallas as pl
from jax.experimental.pallas import tpu as pltpu
from jax.experimental.pallas import tpu_sc as plsc

B = 16384
S = 200
D = 12
V = 50
N = B * S
NC = 2
NS = 16
NW = NC * NS
PER_W = N // NW
CHUNK = 2048
GROUPS = CHUNK // 16
ROUNDS = PER_W // CHUNK
NBUF = 2
LANES = 16
ROWW = D * LANES

def _make_gather_kernel():
    mesh = plsc.VectorSubcoreMesh(core_axis_name="c", subcore_axis_name="s")

    @functools.partial(
        pl.kernel,
        out_type=jax.ShapeDtypeStruct((N * D // 128, 128), jnp.float32),
        mesh=mesh,
        scratch_types=[
            pltpu.VMEM((V * ROWW,), jnp.float32),
            pltpu.VMEM((CHUNK // 128, 128), jnp.int32),
            pltpu.VMEM((CHUNK // 128, 128), jnp.int32),
            pltpu.VMEM((CHUNK * D // 128, 128), jnp.float32),
            pltpu.VMEM((CHUNK * D // 128, 128), jnp.float32),
            pltpu.VMEM_SHARED((NBUF * NS * (CHUNK * D // 128), 128),
                              jnp.float32),
            pltpu.SemaphoreType.DMA,
            pltpu.SemaphoreType.DMA,
            pltpu.SemaphoreType.DMA,
            pltpu.SemaphoreType.DMA,
            pltpu.SemaphoreType.DMA,
        ],
        compiler_params=pltpu.CompilerParams(needs_layout_passes=False),
    )
    def gather_kernel(idx_hbm, tab_hbm, out_hbm, tab_v, idx_v0, idx_v1,
                      rows_v0, rows_v1, shared_v, sem_t, sem_i0, sem_i1,
                      sem_o0, sem_o1):
        idx_bufs = (idx_v0, idx_v1)
        rows_bufs = (rows_v0, rows_v1)
        sems_i = (sem_i0, sem_i1)
        sems_o = (sem_o0, sem_o1)

        cid = lax.axis_index("c")
        sid = lax.axis_index("s")

        pltpu.async_copy(tab_hbm, tab_v, sem_t).wait()

        iota16 = lax.iota(jnp.int32, 16)
        pats = [(iota16 + 16 * m) // D for m in range(D)]
        cves = [((iota16 + 16 * m) % D) * LANES + iota16 for m in range(D)]

        IDXR = CHUNK // 128
        OUTR = CHUNK * D // 128
        SCR = NS * OUTR

        def idx_row0(r):
            return pl.multiple_of(
                r * (NW * CHUNK // 128) + cid * (NS * CHUNK // 128)
                + sid * IDXR, 8)

        def start_idx(r, bf):
            rc = jnp.minimum(r, ROUNDS - 1)
            pltpu.async_copy(
                idx_hbm.at[pl.ds(idx_row0(rc), IDXR)],
                idx_bufs[bf], sems_i[bf])

        def wait_idx(bf):
            pltpu.make_async_copy(
                idx_hbm.at[pl.ds(0, IDXR)], idx_bufs[bf], sems_i[bf]).wait()

        def start_out(r, bf):
            dst0 = pl.multiple_of(r * (NW * OUTR) + cid * SCR, 8)
            pltpu.async_copy(
                shared_v.at[pl.ds(bf * SCR, SCR)],
                out_hbm.at[pl.ds(dst0, SCR)], sems_o[bf])

        def wait_out(bf):
            pltpu.make_async_copy(
                shared_v.at[pl.ds(bf * SCR, SCR)],
                out_hbm.at[pl.ds(0, SCR)], sems_o[bf]).wait()

        start_idx(0, 0)
        start_idx(1, 1)

        def round_pair(r2, carry):
            for bf in range(NBUF):
                r = r2 * NBUF + bf
                wait_idx(bf)

                idx_buf = idx_bufs[bf]
                rows_buf = rows_bufs[bf]

                @plsc.parallel_loop(0, GROUPS, unroll=2)
                def _grp(g):
                    ivec = idx_buf[g // 8, pl.ds((g % 8) * 16, 16)] * ROWW
                    gbase = g * (16 * D)
                    for m in range(D):
                        src = ivec.at[pats[m]].get(mode="promise_in_bounds")
                        vals = plsc.load_gather(tab_v, [src + cves[m]])
                        flat = gbase + m * 16
                        rows_buf[flat // 128, pl.ds(flat % 128, 16)] = vals

                @pl.when(jnp.logical_and(r >= NBUF, sid == 0))
                def _():
                    wait_out(bf)

                plsc.subcore_barrier()
                pltpu.sync_copy(
                    rows_buf,
                    shared_v.at[pl.ds(
                        pl.multiple_of((bf * NS + sid) * OUTR, 8), OUTR)])
                plsc.subcore_barrier()

                @pl.when(sid == 0)
                def _():
                    start_out(r, bf)

                start_idx(r + NBUF, bf)
            return carry

        lax.fori_loop(0, ROUNDS // NBUF, round_pair, 0)

        @pl.when(sid == 0)
        def _():
            for bf in range(NBUF):
                wait_out(bf)

        for bf in range(NBUF):
            wait_idx(bf)
        plsc.subcore_barrier()

    return gather_kernel


_gather = _make_gather_kernel()


def _linear_body(x_ref, w_ref, b_ref, o_ref):
    o_ref[...] = (
        jnp.dot(x_ref[...], w_ref[...], preferred_element_type=jnp.float32)
        + b_ref[...]
    )


def _linear(linear_in, Wt, b2):
    return pl.pallas_call(
        _linear_body,
        out_shape=jax.ShapeDtypeStruct((B, 5), jnp.float32),
        grid=(8,),
        in_specs=[
            pl.BlockSpec((B // 8, 5), lambda i: (i, 0)),
            pl.BlockSpec((5, 5), lambda i: (0, 0)),
            pl.BlockSpec((1, 5), lambda i: (0, 0)),
        ],
        out_specs=pl.BlockSpec((B // 8, 5), lambda i: (i, 0)),
    )(linear_in, Wt, b2)


def kernel(indices, linear_in, emb_table, W, b):
    idx2 = indices.reshape(N // 128, 128).astype(jnp.int32)
    tab_rep = jnp.repeat(emb_table.reshape(V * D), LANES)
    a = _gather(idx2, tab_rep).reshape(B, S, D)
    q = _linear(linear_in, W.T, b.reshape(1, 5))
    return (a, q)


# speedup vs baseline: 1.4554x; 1.4554x over previous
import functools

import jax
import jax.numpy as jnp
from jax import lax
from jax.experimental import pallas as pl
from jax.experimental.pallas import tpu as pltpu
from jax.experimental.pallas import tpu_sc as plsc

B = 16384
S = 200
D = 12
V = 50
N = B * S
NC = 2
NS = 16
NW = NC * NS
PER_W = N // NW
CHUNK = 2048
GROUPS = CHUNK // 16
ROUNDS = PER_W // CHUNK
NBUF = 2
LANES = 16
ROWW = D * LANES

def _make_gather_kernel():
    mesh = plsc.VectorSubcoreMesh(core_axis_name="c", subcore_axis_name="s")

    @functools.partial(
        pl.kernel,
        out_type=jax.ShapeDtypeStruct((N, D), jnp.float32),
        mesh=mesh,
        scratch_types=[
            pltpu.VMEM((V * ROWW,), jnp.float32),
            pltpu.VMEM((CHUNK // 128, 128), jnp.int32),
            pltpu.VMEM((CHUNK // 128, 128), jnp.int32),
            pltpu.VMEM((CHUNK, D), jnp.float32),
            pltpu.VMEM((CHUNK, D), jnp.float32),
            pltpu.SemaphoreType.DMA,
            pltpu.SemaphoreType.DMA,
            pltpu.SemaphoreType.DMA,
            pltpu.SemaphoreType.DMA,
            pltpu.SemaphoreType.DMA,
        ],
        compiler_params=pltpu.CompilerParams(
            use_tc_tiling_on_sc=False, needs_layout_passes=False),
    )
    def gather_kernel(idx_hbm, tab_hbm, out_hbm, tab_v, idx_v0, idx_v1,
                      rows_v0, rows_v1, sem_t, sem_i0, sem_i1,
                      sem_o0, sem_o1):
        idx_bufs = (idx_v0, idx_v1)
        rows_bufs = (rows_v0, rows_v1)
        sems_i = (sem_i0, sem_i1)
        sems_o = (sem_o0, sem_o1)

        wid = lax.axis_index("s") * NC + lax.axis_index("c")
        base_w = wid * PER_W

        pltpu.async_copy(tab_hbm, tab_v, sem_t).wait()

        iota16 = lax.iota(jnp.int32, 16)
        pats = [(iota16 + 16 * m) // D for m in range(D)]
        cves = [((iota16 + 16 * m) % D) * LANES + iota16 for m in range(D)]
        cpat = [(iota16 + 16 * m) % D for m in range(D)]

        IDXR = CHUNK // 128

        def idx_row0(r):
            return pl.multiple_of(base_w // 128 + r * IDXR, 8)

        def start_idx(r, bf):
            rc = jnp.minimum(r, ROUNDS - 1)
            pltpu.async_copy(
                idx_hbm.at[pl.ds(idx_row0(rc), IDXR)],
                idx_bufs[bf], sems_i[bf])

        def wait_idx(bf):
            pltpu.make_async_copy(
                idx_hbm.at[pl.ds(0, IDXR)], idx_bufs[bf], sems_i[bf]).wait()

        def start_out(r, bf):
            dst0 = pl.multiple_of(base_w + r * CHUNK, 8)
            pltpu.async_copy(
                rows_bufs[bf],
                out_hbm.at[pl.ds(dst0, CHUNK)], sems_o[bf])

        def wait_out(bf):
            pltpu.make_async_copy(
                rows_bufs[bf],
                out_hbm.at[pl.ds(0, CHUNK)], sems_o[bf]).wait()

        start_idx(0, 0)
        start_idx(1, 1)

        def round_pair(r2, carry):
            for bf in range(NBUF):
                r = r2 * NBUF + bf
                wait_idx(bf)

                @pl.when(r >= NBUF)
                def _():
                    wait_out(bf)

                idx_buf = idx_bufs[bf]
                rows_buf = rows_bufs[bf]

                @plsc.parallel_loop(0, GROUPS, unroll=2)
                def _grp(g):
                    ivec = idx_buf[g // 8, pl.ds((g % 8) * 16, 16)] * ROWW
                    g16 = g * 16
                    for m in range(D):
                        src = ivec.at[pats[m]].get(mode="promise_in_bounds")
                        vals = plsc.load_gather(tab_v, [src + cves[m]])
                        plsc.store_scatter(
                            rows_buf, [pats[m] + g16, cpat[m]], vals)

                start_out(r, bf)
                start_idx(r + NBUF, bf)
            return carry

        lax.fori_loop(0, ROUNDS // NBUF, round_pair, 0)

        for bf in range(NBUF):
            wait_out(bf)
            wait_idx(bf)

    return gather_kernel


_gather = _make_gather_kernel()


def _linear_body(x_ref, w_ref, b_ref, o_ref):
    o_ref[...] = (
        jnp.dot(x_ref[...], w_ref[...], preferred_element_type=jnp.float32)
        + b_ref[...]
    )


def _linear(linear_in, Wt, b2):
    return pl.pallas_call(
        _linear_body,
        out_shape=jax.ShapeDtypeStruct((B, 5), jnp.float32),
        grid=(8,),
        in_specs=[
            pl.BlockSpec((B // 8, 5), lambda i: (i, 0)),
            pl.BlockSpec((5, 5), lambda i: (0, 0)),
            pl.BlockSpec((1, 5), lambda i: (0, 0)),
        ],
        out_specs=pl.BlockSpec((B // 8, 5), lambda i: (i, 0)),
    )(linear_in, Wt, b2)


def kernel(indices, linear_in, emb_table, W, b):
    idx2 = indices.reshape(N // 128, 128).astype(jnp.int32)
    tab_rep = jnp.repeat(emb_table.reshape(V * D), LANES)
    a = _gather(idx2, tab_rep).reshape(B, S, D)
    q = _linear(linear_in, W.T, b.reshape(1, 5))
    return (a, q)
